# baseline (speedup 1.0000x reference)
import functools

import jax
import jax.numpy as jnp
from jax import lax
from jax.experimental import pallas as pl
from jax.experimental.pallas import tpu as pltpu
from jax.experimental.pallas import tpu_sc as plsc

_KS = 25
_NSLOT = _KS ** 3
_R = 8
_NSUP = (_NSLOT + _R - 1) // _R
_NROWS_PAD = _NSUP * _R
_N = 10000
_E = 160000
_NCORN = 8
_P = _E * _NCORN
_T = 256
_CH = 128
_NWORK = 32
_P_PAD = 1781760
_PER_W = _P_PAD // _NWORK
_NCHUNK = _PER_W // _CH
_NTILE = _P_PAD // _T
_NSTRIPE = _N // 16

_LOWER_CURV = -0.22703196
_LOWER_MYELIN = 1.2585511
_UPPER_CURV = 0.36853024
_UPPER_MYELIN = 1.648841

_LDIMS = [(8, 32), (32, 64), (64, 64), (64, 32), (32, 16)]


def _transform_x(x, r=10.0):
    t0 = (x[:, 0] - _LOWER_CURV) / (_UPPER_CURV - _LOWER_CURV) * (2 * r) - r
    t0 = jnp.where(x[:, 0] > r, r, jnp.where(x[:, 0] < -r, -r, t0))
    t1 = (x[:, 1] - _LOWER_MYELIN) / (_UPPER_MYELIN - _LOWER_MYELIN) * (2 * r) - r
    t1 = jnp.where(x[:, 1] > r, r, jnp.where(x[:, 1] < -r, -r, t1))
    return jnp.stack([t0, t1], axis=1)


def _basis(edge_attr):
    p = jnp.clip(edge_attr, 0.0, 1.0) * (_KS - 1)
    lof = jnp.minimum(jnp.floor(p), _KS - 2.0)
    lo = lof.astype(jnp.int32)
    frac = p - lof
    b01 = jnp.stack([1.0 - frac, frac], axis=-1)
    i01 = jnp.stack([lo, lo + 1], axis=-1)
    bs, ks = [], []
    for c in range(_NCORN):
        c0, c1, c2 = c & 1, (c >> 1) & 1, (c >> 2) & 1
        bs.append(b01[:, 0, c0] * b01[:, 1, c1] * b01[:, 2, c2])
        ks.append(i01[:, 0, c0] + i01[:, 1, c1] * _KS + i01[:, 2, c2] * (_KS * _KS))
    return jnp.stack(bs, axis=1), jnp.stack(ks, axis=1)


def _build_pair_layout(edge_index, edge_attr):
    src = edge_index[0]
    dst = edge_index[1]
    b, k = _basis(edge_attr)
    b_p = b.reshape(-1)
    k_p = k.reshape(-1)
    src_p = jnp.broadcast_to(src[:, None], (_E, _NCORN)).reshape(-1)
    dst_p = jnp.broadcast_to(dst[:, None], (_E, _NCORN)).reshape(-1)

    order = jnp.argsort(k_p)
    k_s = k_p[order]
    sup_s = k_s >> 3
    bounds = jnp.searchsorted(k_s, jnp.arange(_NSUP + 1, dtype=jnp.int32) * _R)
    counts = bounds[1:] - bounds[:-1]
    starts = bounds[:-1]
    padded_counts = ((counts + _T - 1) // _T) * _T
    off_pad = jnp.concatenate([jnp.zeros((1,), jnp.int32),
                               jnp.cumsum(padded_counts).astype(jnp.int32)])
    dest = off_pad[sup_s] + (jnp.arange(_P, dtype=jnp.int32) - starts[sup_s])

    k_pad = jnp.full((_P_PAD,), -(2 ** 20), jnp.int32).at[dest].set(k_s)
    b_pad = jnp.zeros((_P_PAD,), jnp.float32).at[dest].set(b_p[order])
    src_pad = jnp.zeros((_P_PAD,), jnp.int32).at[dest].set(src_p[order])
    dst_pad = jnp.zeros((_P_PAD,), jnp.int32).at[dest].set(dst_p[order])

    tile_sup = jnp.clip(
        jnp.searchsorted(off_pad, jnp.arange(_NTILE, dtype=jnp.int32) * _T,
                         side="right") - 1, 0, _NSUP - 1).astype(jnp.int32)
    return (k_pad.reshape(_P_PAD, 1), b_pad.reshape(_P_PAD, 1),
            src_pad, dst_pad, tile_sup)


def _sc_gather(h, src_pad, fin):
    mesh = plsc.VectorSubcoreMesh(core_axis_name="c", subcore_axis_name="s")

    @functools.partial(
        pl.kernel, mesh=mesh,
        out_type=jax.ShapeDtypeStruct((_P_PAD, fin), jnp.float32),
        compiler_params=pltpu.CompilerParams(use_tc_tiling_on_sc=False),
        scratch_types=[
            pltpu.VMEM((_CH,), jnp.int32),
            pltpu.VMEM((_CH, fin), jnp.float32),
            pltpu.SemaphoreType.DMA,
        ],
    )
    def gk(h_hbm, src_hbm, z_hbm, idx_v, rows_v, sem):
        wid = lax.axis_index("s") * 2 + lax.axis_index("c")
        base0 = wid * _PER_W

        def body(i, carry):
            base = base0 + i * _CH
            pltpu.sync_copy(src_hbm.at[pl.ds(base, _CH)], idx_v)
            pltpu.async_copy(h_hbm.at[idx_v], rows_v, sem).wait()
            pltpu.sync_copy(rows_v, z_hbm.at[pl.ds(base, _CH)])
            return carry

        lax.fori_loop(0, _NCHUNK, body, 0)

    return gk(h, src_pad)


def _tc_window_mm(tile_sup, k_pad, b_pad, z, w_sup, fin, fout):
    def body(sup_ref, k_ref, b_ref, z_ref, w_ref, r_ref):
        t = pl.program_id(0)
        base = sup_ref[t] * _R
        kv = k_ref[...]
        zv = z_ref[...]
        acc = jnp.zeros((_T, fout), jnp.float32)
        for row in range(_R):
            m = kv == (base + row)
            zm = jnp.where(m, zv, 0.0)
            acc = acc + jnp.dot(zm, w_ref[0, row],
                                preferred_element_type=jnp.float32)
        r_ref[...] = acc * b_ref[...]

    grid_spec = pltpu.PrefetchScalarGridSpec(
        num_scalar_prefetch=1,
        grid=(_NTILE,),
        in_specs=[
            pl.BlockSpec((_T, 1), lambda t, sup: (t, 0)),
            pl.BlockSpec((_T, 1), lambda t, sup: (t, 0)),
            pl.BlockSpec((_T, fin), lambda t, sup: (t, 0)),
            pl.BlockSpec((1, _R, fin, fout), lambda t, sup: (sup[t], 0, 0, 0)),
        ],
        out_specs=pl.BlockSpec((_T, fout), lambda t, sup: (t, 0)),
    )
    return pl.pallas_call(
        body, grid_spec=grid_spec,
        out_shape=jax.ShapeDtypeStruct((_P_PAD, fout), jnp.float32),
    )(tile_sup, k_pad, b_pad, z, w_sup)


def _sc_scatter(r, dst_pad, zeros_nf, fout):
    mesh = plsc.VectorSubcoreMesh(core_axis_name="c", subcore_axis_name="s")

    @functools.partial(
        pl.kernel, mesh=mesh,
        out_type=jax.ShapeDtypeStruct((2, _N, fout), jnp.float32),
        compiler_params=pltpu.CompilerParams(use_tc_tiling_on_sc=False),
        scratch_types=[
            pltpu.VMEM((_CH,), jnp.int32),
            pltpu.VMEM((_CH, fout), jnp.float32),
            pltpu.VMEM_SHARED((_N, fout), jnp.float32),
        ],
    )
    def sk(r_hbm, dst_hbm, zeros_hbm, out_hbm, d_v, r_v, acc):
        cid = lax.axis_index("c")
        sid = lax.axis_index("s")
        wid = sid * 2 + cid
        base0 = wid * _PER_W
        stripe = sid * _NSTRIPE
        pltpu.sync_copy(zeros_hbm.at[pl.ds(stripe, _NSTRIPE)],
                        acc.at[pl.ds(stripe, _NSTRIPE)])
        plsc.subcore_barrier()

        def body(i, carry):
            base = base0 + i * _CH
            pltpu.sync_copy(dst_hbm.at[pl.ds(base, _CH)], d_v)
            pltpu.sync_copy(r_hbm.at[pl.ds(base, _CH)], r_v)
            pltpu.sync_copy(r_v, acc.at[d_v], add=True)
            return carry

        lax.fori_loop(0, _NCHUNK, body, 0)
        plsc.subcore_barrier()
        pltpu.sync_copy(acc.at[pl.ds(stripe, _NSTRIPE)],
                        out_hbm.at[cid, pl.ds(stripe, _NSTRIPE)])

    return sk(r, dst_pad, zeros_nf)


def _tc_combine(partials, h, root, bias, fin, fout):
    def body(p_ref, h_ref, root_ref, bias_ref, o_ref):
        tot = (p_ref[0] + p_ref[1]
               + jnp.dot(h_ref[...], root_ref[...],
                         preferred_element_type=jnp.float32)
               + bias_ref[...])
        o_ref[...] = jnp.where(tot > 0, tot, jnp.exp(tot) - 1.0)

    nb = 1000
    return pl.pallas_call(
        body,
        grid=(_N // nb,),
        in_specs=[
            pl.BlockSpec((2, nb, fout), lambda i: (0, i, 0)),
            pl.BlockSpec((nb, fin), lambda i: (i, 0)),
            pl.BlockSpec((fin, fout), lambda i: (0, 0)),
            pl.BlockSpec((1, fout), lambda i: (0, 0)),
        ],
        out_specs=pl.BlockSpec((nb, fout), lambda i: (i, 0)),
        out_shape=jax.ShapeDtypeStruct((_N, fout), jnp.float32),
    )(partials, h, root, bias)


def _pad_weights(W, root, bias, fin, fout):
    fi0, fo0 = W.shape[1], W.shape[2]
    Wp = jnp.pad(W, ((0, _NROWS_PAD - _NSLOT), (0, fin - fi0), (0, fout - fo0)))
    Wp = Wp.reshape(_NSUP, _R, fin, fout)
    rp = jnp.pad(root, ((0, fin - fi0), (0, fout - fo0)))
    bp = jnp.pad(bias, (0, fout - fo0)).reshape(1, fout)
    return Wp, rp, bp


def kernel(x, edge_index, edge_attr, W1, root1, b1, W2, root2, b2, W3, root3,
           b3, W4, root4, b4, W5, root5, b5):
    k_pad, b_pad, src_pad, dst_pad, tile_sup = _build_pair_layout(
        edge_index, edge_attr)
    h = _transform_x(x.astype(jnp.float32))
    h = jnp.pad(h, ((0, 0), (0, _LDIMS[0][0] - 2)))

    params = [(W1, root1, b1), (W2, root2, b2), (W3, root3, b3),
              (W4, root4, b4), (W5, root5, b5)]
    for (W, root, bias), (fin, fout) in zip(params, _LDIMS):
        Wp, rp, bp = _pad_weights(W, root, bias, fin, fout)
        zeros_nf = jnp.zeros((_N, fout), jnp.float32)
        z = _sc_gather(h, src_pad, fin)
        r = _tc_window_mm(tile_sup, k_pad, b_pad, z, Wp, fin, fout)
        partials = _sc_scatter(r, dst_pad, zeros_nf, fout)
        h = _tc_combine(partials, h, rp, bp, fin, fout)
    return h[:, 0]
